# baseline (speedup 1.0000x reference)
import functools

import jax
import jax.numpy as jnp
from jax import lax
from jax.experimental import pallas as pl
from jax.experimental.pallas import tpu as pltpu
from jax.experimental.pallas import tpu_sc as plsc

_RADIUS2 = 0.2 * 0.2
_K = 32
_L = 16
_NC = 2
_NS = 16
_NW = _NC * _NS
_U = 8
_BF16_MASK = -65536


def _bf16_round(v):
    u = plsc.bitcast(v, jnp.int32)
    lsb = lax.shift_right_logical(u, 16) & 1
    r = (u + lsb + 0x7FFF) & _BF16_MASK
    return plsc.bitcast(r, jnp.float32)


def _ball_query(points_f, newxyz_f, B, N, M):
    nslots = _NW // B
    qpw = M // nslots
    nchunk = N // _L
    nstep = nchunk // _U
    mesh = plsc.VectorSubcoreMesh(
        core_axis_name="c", subcore_axis_name="s", num_cores=_NC, num_subcores=_NS
    )

    @functools.partial(
        pl.kernel,
        out_type=jax.ShapeDtypeStruct((B * M * _K,), jnp.int32),
        mesh=mesh,
        scratch_types=[
            pltpu.VMEM((N,), jnp.float32),
            pltpu.VMEM((N,), jnp.float32),
            pltpu.VMEM((N,), jnp.float32),
            pltpu.VMEM((N,), jnp.float32),
            pltpu.VMEM((qpw,), jnp.float32),
            pltpu.VMEM((qpw,), jnp.float32),
            pltpu.VMEM((qpw,), jnp.float32),
            pltpu.VMEM((_K + _U * _L + _L,), jnp.int32),
            pltpu.VMEM((qpw * _K,), jnp.int32),
        ],
        compiler_params=pltpu.CompilerParams(needs_layout_passes=False),
    )
    def kern(points_hbm, newxyz_hbm, idx_out, px, py, pz, pn, qx, qy, qz, buf, acc):
        wid = lax.axis_index("s") * _NC + lax.axis_index("c")
        b = wid // nslots
        m0 = (wid % nslots) * qpw
        pltpu.sync_copy(points_hbm.at[pl.ds((b * 3 + 0) * N, N)], px)
        pltpu.sync_copy(points_hbm.at[pl.ds((b * 3 + 1) * N, N)], py)
        pltpu.sync_copy(points_hbm.at[pl.ds((b * 3 + 2) * N, N)], pz)
        pltpu.sync_copy(newxyz_hbm.at[pl.ds((b * 3 + 0) * M + m0, qpw)], qx)
        pltpu.sync_copy(newxyz_hbm.at[pl.ds((b * 3 + 1) * M + m0, qpw)], qy)
        pltpu.sync_copy(newxyz_hbm.at[pl.ds((b * 3 + 2) * M + m0, qpw)], qz)

        two = jnp.float32(2.0)

        @plsc.parallel_loop(0, nchunk, 1, unroll=8)
        def _prep(j):
            s = pl.ds(j * _L, _L)
            xv = px[s]
            yv = py[s]
            zv = pz[s]
            pn[s] = (xv * xv + yv * yv) + zv * zv
            px[s] = two * _bf16_round(xv)
            py[s] = two * _bf16_round(yv)
            pz[s] = two * _bf16_round(zv)

        iota = lax.iota(jnp.int32, _L)
        r2 = jnp.float32(_RADIUS2)

        def per_query(qi, _):
            qsel = jnp.full((_L,), qi, jnp.int32)
            qxv = plsc.load_gather(qx, [qsel])
            qyv = plsc.load_gather(qy, [qsel])
            qzv = plsc.load_gather(qz, [qsel])
            qn = (qxv * qxv + qyv * qyv) + qzv * qzv
            qxv = _bf16_round(qxv)
            qyv = _bf16_round(qyv)
            qzv = _bf16_round(qzv)
            buf[pl.ds(0, _L)] = jnp.zeros((_L,), jnp.int32)

            def cond(c):
                return (c[0] < nstep) & (c[1][0] < _K)

            def step(c):
                j, cntv = c
                base = j * (_U * _L)
                for u in range(_U):
                    s = pl.ds(base + u * _L, _L)
                    cross2 = (qxv * px[s] + qyv * py[s]) + qzv * pz[s]
                    d2 = (qn + pn[s]) - cross2
                    msk = d2 <= r2
                    rank = plsc.cumsum(msk.astype(jnp.int32))
                    plsc.store_scatter(
                        buf, [(cntv + rank) - 1], iota + (base + u * _L), mask=msk
                    )
                    cntv = cntv + plsc.all_reduce_population_count(msk)
                return j + jnp.int32(1), cntv

            _, cntv = lax.while_loop(
                cond, step, (jnp.int32(0), jnp.zeros((_L,), jnp.int32))
            )
            lastv = jnp.maximum(cntv - 1, 0)
            sel0 = plsc.load_gather(buf, [jnp.minimum(iota, lastv)])
            sel1 = plsc.load_gather(buf, [jnp.minimum(iota + _L, lastv)])
            acc[pl.ds(qi * _K, _L)] = sel0
            acc[pl.ds(qi * _K + _L, _L)] = sel1
            return 0

        lax.fori_loop(0, qpw, per_query, 0)
        pltpu.sync_copy(acc, idx_out.at[pl.ds((b * M + m0) * _K, qpw * _K)])

    return kern(points_f, newxyz_f)


def _grouped_gather(features_f, points_f, newxyz_f, idx, B, C, N, M):
    CH = C + 3
    nslots = _NW // B
    cpw = C // nslots
    total = M * _K
    chunk = 8192
    nch = total // chunk
    nvec = chunk // _L
    kshift = (_K - 1).bit_length()
    mesh = plsc.VectorSubcoreMesh(
        core_axis_name="c", subcore_axis_name="s", num_cores=_NC, num_subcores=_NS
    )

    @functools.partial(
        pl.kernel,
        out_type=jax.ShapeDtypeStruct((B * CH * total,), jnp.float32),
        mesh=mesh,
        scratch_types=[
            pltpu.VMEM((total,), jnp.int32),
            pltpu.VMEM((N,), jnp.float32),
            pltpu.VMEM((N,), jnp.float32),
            pltpu.VMEM((chunk,), jnp.float32),
            pltpu.VMEM((chunk,), jnp.float32),
            pltpu.VMEM((M,), jnp.float32),
            pltpu.SemaphoreType.DMA,
            pltpu.SemaphoreType.DMA,
        ],
        compiler_params=pltpu.CompilerParams(needs_layout_passes=False),
    )
    def kern(feat_hbm, pts_hbm, ctr_hbm, idx_hbm, out,
             idxb, row0, row1, ob0, ob1, ctr, sem_row, sem_out):
        wid = lax.axis_index("s") * _NC + lax.axis_index("c")
        b = wid // nslots
        slot = wid % nslots
        rows = (row0, row1)
        obs = (ob0, ob1)
        nxyz = 3
        iota = lax.iota(jnp.int32, _L)

        pltpu.sync_copy(idx_hbm.at[pl.ds(b * total, total)], idxb)
        pltpu.async_copy(
            feat_hbm.at[pl.ds((b * C + slot * cpw) * N, N)], row0, sem_row
        ).wait()
        pltpu.async_copy(
            feat_hbm.at[pl.ds((b * C + slot * cpw + 1) * N, N)], row1, sem_row
        )

        nout = 0

        for cc in range(cpw):
            row = rows[cc % 2]
            ch = slot * cpw + cc
            obase = (b * CH + nxyz + ch) * total
            if cc > 0:
                pltpu.make_async_copy(
                    feat_hbm.at[pl.ds(0, N)], row, sem_row
                ).wait()
            for ck in range(nch):
                ob = obs[ck % 2]
                if nout >= 2:
                    pltpu.make_async_copy(ob, out.at[pl.ds(0, chunk)], sem_out).wait()
                    nout -= 1

                @plsc.parallel_loop(0, nvec, 1, unroll=8)
                def _g(j, _ck=ck, _ob=ob, _row=row):
                    p = _ck * chunk + j * _L
                    idxv = idxb[pl.ds(p, _L)]
                    _ob[pl.ds(j * _L, _L)] = plsc.load_gather(_row, [idxv])

                pltpu.async_copy(ob, out.at[pl.ds(obase + ck * chunk, chunk)], sem_out)
                nout += 1
            if cc + 1 < cpw:
                if cc + 2 < cpw:
                    pltpu.async_copy(
                        feat_hbm.at[pl.ds((b * C + slot * cpw + cc + 2) * N, N)],
                        rows[cc % 2],
                        sem_row,
                    )

        for _ in range(nout):
            pltpu.make_async_copy(ob0, out.at[pl.ds(0, chunk)], sem_out).wait()

        @pl.when(slot < nxyz)
        def _():
            pltpu.sync_copy(pts_hbm.at[pl.ds((b * 3 + slot) * N, N)], row0)
            pltpu.sync_copy(ctr_hbm.at[pl.ds((b * 3 + slot) * M, M)], ctr)
            obase = (b * CH + slot) * total
            xout = 0
            for ck in range(nch):
                ob = obs[ck % 2]
                if ck >= 2:
                    pltpu.make_async_copy(ob, out.at[pl.ds(0, chunk)], sem_out).wait()

                @plsc.parallel_loop(0, nvec, 1, unroll=8)
                def _g(j, _ck=ck, _ob=ob):
                    p = _ck * chunk + j * _L
                    idxv = idxb[pl.ds(p, _L)]
                    v = plsc.load_gather(row0, [idxv])
                    mv = lax.shift_right_logical(iota + p, kshift)
                    cv = plsc.load_gather(ctr, [mv])
                    _ob[pl.ds(j * _L, _L)] = v - cv

                pltpu.async_copy(ob, out.at[pl.ds(obase + ck * chunk, chunk)], sem_out)
            for ck in range(min(nch, 2)):
                pltpu.make_async_copy(ob0, out.at[pl.ds(0, chunk)], sem_out).wait()

    return kern(features_f, points_f, newxyz_f, idx)


def kernel(points_xyz, new_xyz, features):
    B, N, _ = points_xyz.shape
    M = new_xyz.shape[1]
    C = features.shape[1]
    points_f = jnp.transpose(points_xyz, (0, 2, 1)).reshape(-1)
    newxyz_f = jnp.transpose(new_xyz, (0, 2, 1)).reshape(-1)
    features_f = features.reshape(-1)
    idx = _ball_query(points_f, newxyz_f, B, N, M)
    out = _grouped_gather(features_f, points_f, newxyz_f, idx, B, C, N, M)
    return out.reshape(B, C + 3, M, _K)


# speedup vs baseline: 1.3258x; 1.3258x over previous
import functools

import jax
import jax.numpy as jnp
from jax import lax
from jax.experimental import pallas as pl
from jax.experimental.pallas import tpu as pltpu
from jax.experimental.pallas import tpu_sc as plsc

_RADIUS2 = 0.2 * 0.2
_K = 32
_L = 16
_NC = 2
_NS = 16
_U = 16
_BF16_MASK = -65536


def _bf16_round(v):
    u = plsc.bitcast(v, jnp.int32)
    lsb = lax.shift_right_logical(u, 16) & 1
    r = (u + lsb + 0x7FFF) & _BF16_MASK
    return plsc.bitcast(r, jnp.float32)


def _fused(points_f, newxyz_f, features_f, B, C, N, M):
    CH = C + 3
    qpw = M // _NS
    cpw = C // _NS
    nchunk = N // _L
    nstep = nchunk // _U
    total = M * _K
    nrow = total // N
    chunk = 8192
    nch = total // chunk
    nvec = chunk // _L
    kshift = (_K - 1).bit_length()
    mesh = plsc.VectorSubcoreMesh(
        core_axis_name="c", subcore_axis_name="s", num_cores=_NC, num_subcores=_NS
    )

    @functools.partial(
        pl.kernel,
        out_type=(
            jax.ShapeDtypeStruct((B * total,), jnp.int32),
            jax.ShapeDtypeStruct((B * CH * total,), jnp.float32),
        ),
        mesh=mesh,
        scratch_types=[
            pltpu.VMEM((N,), jnp.int32),
            pltpu.VMEM((N,), jnp.int32),
            pltpu.VMEM((N,), jnp.int32),
            pltpu.VMEM((N,), jnp.int32),
            pltpu.VMEM((qpw,), jnp.float32),
            pltpu.VMEM((qpw,), jnp.float32),
            pltpu.VMEM((qpw,), jnp.float32),
            pltpu.VMEM((_K + _U * _L + _L,), jnp.int32),
            pltpu.VMEM((qpw * _K,), jnp.int32),
            pltpu.VMEM((N,), jnp.float32),
            pltpu.VMEM((N,), jnp.float32),
            pltpu.VMEM((chunk,), jnp.float32),
            pltpu.VMEM((chunk,), jnp.float32),
            pltpu.VMEM((M,), jnp.float32),
            pltpu.SemaphoreType.DMA,
            pltpu.SemaphoreType.DMA,
        ],
        compiler_params=pltpu.CompilerParams(needs_layout_passes=False),
    )
    def kern(points_hbm, pointsi_hbm, newxyz_hbm, feat_hbm, idx_hbm, out,
             px, py, pz, pn, qx, qy, qz, buf, acc,
             row0, row1, ob0, ob1, ctr, sem_row, sem_out):
        b = lax.axis_index("c")
        slot = lax.axis_index("s")

        m0 = slot * qpw
        pltpu.sync_copy(pointsi_hbm.at[pl.ds((b * 3 + 0) * N, N)], px)
        pltpu.sync_copy(pointsi_hbm.at[pl.ds((b * 3 + 1) * N, N)], py)
        pltpu.sync_copy(pointsi_hbm.at[pl.ds((b * 3 + 2) * N, N)], pz)
        pltpu.sync_copy(newxyz_hbm.at[pl.ds((b * 3 + 0) * M + m0, qpw)], qx)
        pltpu.sync_copy(newxyz_hbm.at[pl.ds((b * 3 + 1) * M + m0, qpw)], qy)
        pltpu.sync_copy(newxyz_hbm.at[pl.ds((b * 3 + 2) * M + m0, qpw)], qz)

        two = jnp.float32(2.0)

        @plsc.parallel_loop(0, nchunk, 1, unroll=8)
        def _prep(j):
            s = pl.ds(j * _L, _L)
            xv = plsc.bitcast(px[s], jnp.float32)
            yv = plsc.bitcast(py[s], jnp.float32)
            zv = plsc.bitcast(pz[s], jnp.float32)
            pn[s] = plsc.bitcast((xv * xv + yv * yv) + zv * zv, jnp.int32)
            px[s] = plsc.bitcast(two * _bf16_round(xv), jnp.int32)
            py[s] = plsc.bitcast(two * _bf16_round(yv), jnp.int32)
            pz[s] = plsc.bitcast(two * _bf16_round(zv), jnp.int32)

        iota = lax.iota(jnp.int32, _L)
        r2 = jnp.float32(_RADIUS2)

        def per_query(qi, _):
            qsel = jnp.full((_L,), qi, jnp.int32)
            qxv = plsc.load_gather(qx, [qsel])
            qyv = plsc.load_gather(qy, [qsel])
            qzv = plsc.load_gather(qz, [qsel])
            qn = (qxv * qxv + qyv * qyv) + qzv * qzv
            qxv = _bf16_round(qxv)
            qyv = _bf16_round(qyv)
            qzv = _bf16_round(qzv)
            buf[pl.ds(0, _L)] = jnp.zeros((_L,), jnp.int32)

            def cond(c):
                return (c[0] < nstep) & (c[1][0] < _K)

            def step(c):
                j, cntv0 = c
                base = j * _U

                @plsc.parallel_loop(0, _U, 1, unroll=8, carry=cntv0)
                def cntv(u, cv):
                    ch = base + u
                    s = pl.ds(ch * _L, _L)
                    xv = plsc.bitcast(px[s], jnp.float32)
                    yv = plsc.bitcast(py[s], jnp.float32)
                    zv = plsc.bitcast(pz[s], jnp.float32)
                    nv = plsc.bitcast(pn[s], jnp.float32)
                    cross2 = (qxv * xv + qyv * yv) + qzv * zv
                    d2 = (qn + nv) - cross2
                    msk = d2 <= r2
                    rank = plsc.cumsum(msk.astype(jnp.int32))
                    plsc.store_scatter(
                        buf, [(cv + rank) - 1], iota + ch * _L, mask=msk
                    )
                    return cv + plsc.all_reduce_population_count(msk)

                return j + jnp.int32(1), cntv

            _, cntv = lax.while_loop(
                cond, step, (jnp.int32(0), jnp.zeros((_L,), jnp.int32))
            )
            lastv = jnp.maximum(cntv - 1, 0)
            sel0 = plsc.load_gather(buf, [jnp.minimum(iota, lastv)])
            sel1 = plsc.load_gather(buf, [jnp.minimum(iota + _L, lastv)])
            acc[pl.ds(qi * _K, _L)] = sel0
            acc[pl.ds(qi * _K + _L, _L)] = sel1
            return 0

        lax.fori_loop(0, qpw, per_query, 0)
        pltpu.sync_copy(acc, idx_hbm.at[pl.ds(b * total + m0 * _K, qpw * _K)])

        plsc.subcore_barrier()

        prow = (px, py, pz, pn)
        for r in range(nrow):
            pltpu.sync_copy(idx_hbm.at[pl.ds(b * total + r * N, N)], prow[r])

        rows = (row0, row1)
        obs = (ob0, ob1)
        nxyz = 3

        pltpu.async_copy(
            feat_hbm.at[pl.ds((b * C + slot * cpw) * N, N)], row0, sem_row
        ).wait()
        pltpu.async_copy(
            feat_hbm.at[pl.ds((b * C + slot * cpw + 1) * N, N)], row1, sem_row
        )

        nout = 0

        for cc in range(cpw):
            row = rows[cc % 2]
            ch = slot * cpw + cc
            obase = (b * CH + nxyz + ch) * total
            if cc > 0:
                pltpu.make_async_copy(
                    feat_hbm.at[pl.ds(0, N)], row, sem_row
                ).wait()
            for ck in range(nch):
                ob = obs[ck % 2]
                irow = prow[ck // 2]
                ioff = (ck % 2) * chunk
                if nout >= 2:
                    pltpu.make_async_copy(ob, out.at[pl.ds(0, chunk)], sem_out).wait()
                    nout -= 1

                @plsc.parallel_loop(0, nvec, 1, unroll=8)
                def _g(j, _ioff=ioff, _ob=ob, _row=row, _irow=irow):
                    idxv = _irow[pl.ds(_ioff + j * _L, _L)]
                    _ob[pl.ds(j * _L, _L)] = plsc.load_gather(_row, [idxv])

                pltpu.async_copy(ob, out.at[pl.ds(obase + ck * chunk, chunk)], sem_out)
                nout += 1
            if cc + 2 < cpw:
                pltpu.async_copy(
                    feat_hbm.at[pl.ds((b * C + slot * cpw + cc + 2) * N, N)],
                    rows[cc % 2],
                    sem_row,
                )

        for _ in range(nout):
            pltpu.make_async_copy(ob0, out.at[pl.ds(0, chunk)], sem_out).wait()

        @pl.when(slot < nxyz)
        def _():
            pltpu.sync_copy(points_hbm.at[pl.ds((b * 3 + slot) * N, N)], row0)
            pltpu.sync_copy(newxyz_hbm.at[pl.ds((b * 3 + slot) * M, M)], ctr)
            obase = (b * CH + slot) * total
            for ck in range(nch):
                ob = obs[ck % 2]
                irow = prow[ck // 2]
                ioff = (ck % 2) * chunk
                if ck >= 2:
                    pltpu.make_async_copy(ob, out.at[pl.ds(0, chunk)], sem_out).wait()

                @plsc.parallel_loop(0, nvec, 1, unroll=8)
                def _g(j, _ck=ck, _ioff=ioff, _ob=ob, _irow=irow):
                    p = _ck * chunk + j * _L
                    idxv = _irow[pl.ds(_ioff + j * _L, _L)]
                    v = plsc.load_gather(row0, [idxv])
                    mv = lax.shift_right_logical(iota + p, kshift)
                    cv = plsc.load_gather(ctr, [mv])
                    _ob[pl.ds(j * _L, _L)] = v - cv

                pltpu.async_copy(ob, out.at[pl.ds(obase + ck * chunk, chunk)], sem_out)
            for _ in range(min(nch, 2)):
                pltpu.make_async_copy(ob0, out.at[pl.ds(0, chunk)], sem_out).wait()

    points_i = lax.bitcast_convert_type(points_f, jnp.int32)
    return kern(points_f, points_i, newxyz_f, features_f)


def kernel(points_xyz, new_xyz, features):
    B, N, _ = points_xyz.shape
    M = new_xyz.shape[1]
    C = features.shape[1]
    points_f = jnp.transpose(points_xyz, (0, 2, 1)).reshape(-1)
    newxyz_f = jnp.transpose(new_xyz, (0, 2, 1)).reshape(-1)
    features_f = features.reshape(-1)
    _, out = _fused(points_f, newxyz_f, features_f, B, C, N, M)
    return out.reshape(B, C + 3, M, _K)


# speedup vs baseline: 3.1681x; 2.3897x over previous
import functools

import jax
import jax.numpy as jnp
from jax import lax
from jax.experimental import pallas as pl
from jax.experimental.pallas import tpu as pltpu
from jax.experimental.pallas import tpu_sc as plsc

_RADIUS2 = 0.2 * 0.2
_K = 32
_L = 16
_NC = 2
_NS = 16
_U = 16
_BF16_MASK = -65536


def _bf16_round(v):
    u = plsc.bitcast(v, jnp.int32)
    lsb = lax.shift_right_logical(u, 16) & 1
    r = (u + lsb + 0x7FFF) & _BF16_MASK
    return plsc.bitcast(r, jnp.float32)


def _fused(points_f, newxyz_f, features_f, B, C, N, M):
    CH = C + 3
    qpw = M // _NS
    cpw = C // _NS
    nchunk = N // _L
    nstep = nchunk // _U
    total = M * _K
    nrow = total // N
    chunk = 8192
    nch = total // chunk
    nvec = chunk // _L
    kshift = (_K - 1).bit_length()
    mesh = plsc.VectorSubcoreMesh(
        core_axis_name="c", subcore_axis_name="s", num_cores=_NC, num_subcores=_NS
    )

    @functools.partial(
        pl.kernel,
        out_type=(
            jax.ShapeDtypeStruct((B * total,), jnp.int32),
            jax.ShapeDtypeStruct((B * CH * total,), jnp.float32),
        ),
        mesh=mesh,
        scratch_types=[
            pltpu.VMEM((N,), jnp.int32),
            pltpu.VMEM((N,), jnp.int32),
            pltpu.VMEM((N,), jnp.int32),
            pltpu.VMEM((N,), jnp.int32),
            pltpu.VMEM((qpw,), jnp.float32),
            pltpu.VMEM((qpw,), jnp.float32),
            pltpu.VMEM((qpw,), jnp.float32),
            pltpu.VMEM((_K + _U * _L + _L,), jnp.int32),
            pltpu.VMEM((qpw * _K,), jnp.int32),
            pltpu.VMEM((N,), jnp.float32),
            pltpu.VMEM((N,), jnp.float32),
            pltpu.VMEM((chunk,), jnp.float32),
            pltpu.VMEM((chunk,), jnp.float32),
            pltpu.VMEM((M,), jnp.float32),
            pltpu.SemaphoreType.DMA,
            pltpu.SemaphoreType.DMA,
        ],
        compiler_params=pltpu.CompilerParams(needs_layout_passes=False),
    )
    def kern(points_hbm, pointsi_hbm, newxyz_hbm, feat_hbm, idx_hbm, out,
             px, py, pz, pn, qx, qy, qz, buf, acc,
             row0, row1, ob0, ob1, ctr, sem_row, sem_out):
        b = lax.axis_index("c")
        slot = lax.axis_index("s")

        m0 = slot * qpw
        pltpu.sync_copy(pointsi_hbm.at[pl.ds((b * 3 + 0) * N, N)], px)
        pltpu.sync_copy(pointsi_hbm.at[pl.ds((b * 3 + 1) * N, N)], py)
        pltpu.sync_copy(pointsi_hbm.at[pl.ds((b * 3 + 2) * N, N)], pz)
        pltpu.sync_copy(newxyz_hbm.at[pl.ds((b * 3 + 0) * M + m0, qpw)], qx)
        pltpu.sync_copy(newxyz_hbm.at[pl.ds((b * 3 + 1) * M + m0, qpw)], qy)
        pltpu.sync_copy(newxyz_hbm.at[pl.ds((b * 3 + 2) * M + m0, qpw)], qz)

        two = jnp.float32(2.0)

        @plsc.parallel_loop(0, nchunk, 1, unroll=8)
        def _prep(j):
            s = pl.ds(j * _L, _L)
            xv = plsc.bitcast(px[s], jnp.float32)
            yv = plsc.bitcast(py[s], jnp.float32)
            zv = plsc.bitcast(pz[s], jnp.float32)
            pn[s] = plsc.bitcast((xv * xv + yv * yv) + zv * zv, jnp.int32)
            px[s] = plsc.bitcast(two * _bf16_round(xv), jnp.int32)
            py[s] = plsc.bitcast(two * _bf16_round(yv), jnp.int32)
            pz[s] = plsc.bitcast(two * _bf16_round(zv), jnp.int32)

        iota = lax.iota(jnp.int32, _L)
        r2 = jnp.float32(_RADIUS2)

        def per_query(qi, _):
            qsel = jnp.full((_L,), qi, jnp.int32)
            qxv = plsc.load_gather(qx, [qsel])
            qyv = plsc.load_gather(qy, [qsel])
            qzv = plsc.load_gather(qz, [qsel])
            qn = (qxv * qxv + qyv * qyv) + qzv * qzv
            qxv = _bf16_round(qxv)
            qyv = _bf16_round(qyv)
            qzv = _bf16_round(qzv)
            buf[pl.ds(0, _L)] = jnp.zeros((_L,), jnp.int32)

            def cond(c):
                return (c[0] < nstep) & (c[1][0] < _K)

            def step(c):
                j, cntv0 = c
                base = j * _U

                @plsc.parallel_loop(0, _U, 1, unroll=8, carry=cntv0)
                def cntv(u, cv):
                    ch = base + u
                    s = pl.ds(ch * _L, _L)
                    xv = plsc.bitcast(px[s], jnp.float32)
                    yv = plsc.bitcast(py[s], jnp.float32)
                    zv = plsc.bitcast(pz[s], jnp.float32)
                    nv = plsc.bitcast(pn[s], jnp.float32)
                    cross2 = (qxv * xv + qyv * yv) + qzv * zv
                    d2 = (qn + nv) - cross2
                    msk = d2 <= r2
                    rank = plsc.cumsum(msk.astype(jnp.int32))
                    plsc.store_scatter(
                        buf, [(cv + rank) - 1], iota + ch * _L, mask=msk
                    )
                    return cv + plsc.all_reduce_population_count(msk)

                return j + jnp.int32(1), cntv

            _, cntv = lax.while_loop(
                cond, step, (jnp.int32(0), jnp.zeros((_L,), jnp.int32))
            )
            lastv = jnp.maximum(cntv - 1, 0)
            sel0 = plsc.load_gather(buf, [jnp.minimum(iota, lastv)])
            sel1 = plsc.load_gather(buf, [jnp.minimum(iota + _L, lastv)])
            kt = lax.shift_right_logical(iota, 3)
            si = iota & 7
            pos = kt * (8 * qpw) + si * qpw + qi
            plsc.store_scatter(acc, [pos], sel0)
            plsc.store_scatter(acc, [pos + 2 * (8 * qpw)], sel1)
            return 0

        lax.fori_loop(0, qpw, per_query, 0)
        blk = 8 * qpw
        for kt_i in range(_K // 8):
            pltpu.sync_copy(
                acc.at[pl.ds(kt_i * blk, blk)],
                idx_hbm.at[pl.ds(b * total + kt_i * (M * 8) + slot * blk, blk)],
            )

        plsc.subcore_barrier()

        prow = (px, py, pz, pn)
        for r in range(nrow):
            pltpu.sync_copy(idx_hbm.at[pl.ds(b * total + r * N, N)], prow[r])

        rows = (row0, row1)
        obs = (ob0, ob1)
        nxyz = 3

        pltpu.async_copy(
            feat_hbm.at[pl.ds((b * C + slot * cpw) * N, N)], row0, sem_row
        ).wait()
        pltpu.async_copy(
            feat_hbm.at[pl.ds((b * C + slot * cpw + 1) * N, N)], row1, sem_row
        )

        nout = 0

        for cc in range(cpw):
            row = rows[cc % 2]
            ch = slot * cpw + cc
            obase = (b * CH + nxyz + ch) * total
            if cc > 0:
                pltpu.make_async_copy(
                    feat_hbm.at[pl.ds(0, N)], row, sem_row
                ).wait()
            for ck in range(nch):
                ob = obs[ck % 2]
                irow = prow[ck // 2]
                ioff = (ck % 2) * chunk
                if nout >= 2:
                    pltpu.make_async_copy(ob, out.at[pl.ds(0, chunk)], sem_out).wait()
                    nout -= 1

                @plsc.parallel_loop(0, nvec, 1, unroll=8)
                def _g(j, _ioff=ioff, _ob=ob, _row=row, _irow=irow):
                    idxv = _irow[pl.ds(_ioff + j * _L, _L)]
                    _ob[pl.ds(j * _L, _L)] = plsc.load_gather(_row, [idxv])

                pltpu.async_copy(ob, out.at[pl.ds(obase + ck * chunk, chunk)], sem_out)
                nout += 1
            if cc + 2 < cpw:
                pltpu.async_copy(
                    feat_hbm.at[pl.ds((b * C + slot * cpw + cc + 2) * N, N)],
                    rows[cc % 2],
                    sem_row,
                )

        for _ in range(nout):
            pltpu.make_async_copy(ob0, out.at[pl.ds(0, chunk)], sem_out).wait()

        @pl.when(slot < nxyz)
        def _():
            pltpu.sync_copy(points_hbm.at[pl.ds((b * 3 + slot) * N, N)], row0)
            pltpu.sync_copy(newxyz_hbm.at[pl.ds((b * 3 + slot) * M, M)], ctr)
            obase = (b * CH + slot) * total
            for ck in range(nch):
                ob = obs[ck % 2]
                irow = prow[ck // 2]
                ioff = (ck % 2) * chunk
                if ck >= 2:
                    pltpu.make_async_copy(ob, out.at[pl.ds(0, chunk)], sem_out).wait()

                @plsc.parallel_loop(0, nvec, 1, unroll=8)
                def _g(j, _ck=ck, _ioff=ioff, _ob=ob, _irow=irow):
                    p = _ck * chunk + j * _L
                    idxv = _irow[pl.ds(_ioff + j * _L, _L)]
                    v = plsc.load_gather(row0, [idxv])
                    pv = iota + p
                    mv = ((lax.shift_right_logical(pv, 10) & 15) * 128) + (pv & 127)
                    cv = plsc.load_gather(ctr, [mv])
                    _ob[pl.ds(j * _L, _L)] = v - cv

                pltpu.async_copy(ob, out.at[pl.ds(obase + ck * chunk, chunk)], sem_out)
            for _ in range(min(nch, 2)):
                pltpu.make_async_copy(ob0, out.at[pl.ds(0, chunk)], sem_out).wait()

    points_i = lax.bitcast_convert_type(points_f, jnp.int32)
    return kern(points_f, points_i, newxyz_f, features_f)


def kernel(points_xyz, new_xyz, features):
    B, N, _ = points_xyz.shape
    M = new_xyz.shape[1]
    C = features.shape[1]
    points_f = jnp.transpose(points_xyz, (0, 2, 1)).reshape(-1)
    newxyz_f = jnp.transpose(new_xyz, (0, 2, 1)).reshape(-1)
    features_f = features.reshape(-1)
    _, out = _fused(points_f, newxyz_f, features_f, B, C, N, M)
    out5 = out.reshape(B, C + 3, _K // 8, M // 128, 8, 128)
    return jnp.transpose(out5, (0, 1, 3, 5, 2, 4)).reshape(B, C + 3, M, _K)


# speedup vs baseline: 3.2075x; 1.0124x over previous
import functools

import jax
import jax.numpy as jnp
from jax import lax
from jax.experimental import pallas as pl
from jax.experimental.pallas import tpu as pltpu
from jax.experimental.pallas import tpu_sc as plsc

_RADIUS2 = 0.2 * 0.2
_K = 32
_L = 16
_NC = 2
_NS = 16
_U = 16
_BF16_MASK = -65536


def _bf16_round(v):
    u = plsc.bitcast(v, jnp.int32)
    lsb = lax.shift_right_logical(u, 16) & 1
    r = (u + lsb + 0x7FFF) & _BF16_MASK
    return plsc.bitcast(r, jnp.float32)


def _fused(points_f, newxyz_f, features_f, B, C, N, M):
    CH = C + 3
    qpw = M // _NS
    cpw = C // _NS
    nchunk = N // _L
    nstep = nchunk // _U
    total = M * _K
    nrow = total // N
    chunk = 8192
    nch = total // chunk
    nvec = chunk // _L
    kshift = (_K - 1).bit_length()
    mesh = plsc.VectorSubcoreMesh(
        core_axis_name="c", subcore_axis_name="s", num_cores=_NC, num_subcores=_NS
    )

    @functools.partial(
        pl.kernel,
        out_type=(
            jax.ShapeDtypeStruct((B * total,), jnp.int32),
            jax.ShapeDtypeStruct((B * CH * total,), jnp.float32),
        ),
        mesh=mesh,
        scratch_types=[
            pltpu.VMEM((N,), jnp.int32),
            pltpu.VMEM((N,), jnp.int32),
            pltpu.VMEM((N,), jnp.int32),
            pltpu.VMEM((N,), jnp.int32),
            pltpu.VMEM((qpw,), jnp.float32),
            pltpu.VMEM((qpw,), jnp.float32),
            pltpu.VMEM((qpw,), jnp.float32),
            pltpu.VMEM((_K + _U * _L + _L,), jnp.int32),
            pltpu.VMEM((qpw * _K,), jnp.int32),
            pltpu.VMEM((N,), jnp.float32),
            pltpu.VMEM((N,), jnp.float32),
            pltpu.VMEM((chunk,), jnp.float32),
            pltpu.VMEM((chunk,), jnp.float32),
            pltpu.VMEM((M,), jnp.float32),
            pltpu.SemaphoreType.DMA,
            pltpu.SemaphoreType.DMA,
            pltpu.SemaphoreType.DMA,
        ],
        compiler_params=pltpu.CompilerParams(needs_layout_passes=False),
    )
    def kern(points_hbm, pointsi_hbm, newxyz_hbm, feat_hbm, idx_hbm, out,
             px, py, pz, pn, qx, qy, qz, buf, acc,
             row0, row1, ob0, ob1, ctr, sem_row, sem_out, sem_pre):
        b = lax.axis_index("c")
        slot = lax.axis_index("s")

        m0 = slot * qpw
        cps = [
            pltpu.async_copy(pointsi_hbm.at[pl.ds((b * 3 + 0) * N, N)], px, sem_row),
            pltpu.async_copy(pointsi_hbm.at[pl.ds((b * 3 + 1) * N, N)], py, sem_row),
            pltpu.async_copy(pointsi_hbm.at[pl.ds((b * 3 + 2) * N, N)], pz, sem_row),
            pltpu.async_copy(newxyz_hbm.at[pl.ds((b * 3 + 0) * M + m0, qpw)], qx, sem_row),
            pltpu.async_copy(newxyz_hbm.at[pl.ds((b * 3 + 1) * M + m0, qpw)], qy, sem_row),
            pltpu.async_copy(newxyz_hbm.at[pl.ds((b * 3 + 2) * M + m0, qpw)], qz, sem_row),
        ]
        pltpu.async_copy(
            feat_hbm.at[pl.ds((b * C + slot * cpw) * N, N)], row0, sem_pre
        )
        pltpu.async_copy(
            feat_hbm.at[pl.ds((b * C + slot * cpw + 1) * N, N)], row1, sem_pre
        )
        for cp in cps:
            cp.wait()

        two = jnp.float32(2.0)

        @plsc.parallel_loop(0, nchunk, 1, unroll=8)
        def _prep(j):
            s = pl.ds(j * _L, _L)
            xv = plsc.bitcast(px[s], jnp.float32)
            yv = plsc.bitcast(py[s], jnp.float32)
            zv = plsc.bitcast(pz[s], jnp.float32)
            pn[s] = plsc.bitcast((xv * xv + yv * yv) + zv * zv, jnp.int32)
            px[s] = plsc.bitcast(two * _bf16_round(xv), jnp.int32)
            py[s] = plsc.bitcast(two * _bf16_round(yv), jnp.int32)
            pz[s] = plsc.bitcast(two * _bf16_round(zv), jnp.int32)

        iota = lax.iota(jnp.int32, _L)
        r2 = jnp.float32(_RADIUS2)

        def per_query(qi, _):
            qsel = jnp.full((_L,), qi, jnp.int32)
            qxv = plsc.load_gather(qx, [qsel])
            qyv = plsc.load_gather(qy, [qsel])
            qzv = plsc.load_gather(qz, [qsel])
            qn = (qxv * qxv + qyv * qyv) + qzv * qzv
            qxv = _bf16_round(qxv)
            qyv = _bf16_round(qyv)
            qzv = _bf16_round(qzv)
            buf[pl.ds(0, _L)] = jnp.zeros((_L,), jnp.int32)

            def cond(c):
                return (c[0] < nstep) & (c[1][0] < _K)

            def step(c):
                j, cntv0 = c
                base = j * _U

                @plsc.parallel_loop(0, _U, 1, unroll=8, carry=cntv0)
                def cntv(u, cv):
                    ch = base + u
                    s = pl.ds(ch * _L, _L)
                    xv = plsc.bitcast(px[s], jnp.float32)
                    yv = plsc.bitcast(py[s], jnp.float32)
                    zv = plsc.bitcast(pz[s], jnp.float32)
                    nv = plsc.bitcast(pn[s], jnp.float32)
                    cross2 = (qxv * xv + qyv * yv) + qzv * zv
                    d2 = (qn + nv) - cross2
                    msk = d2 <= r2
                    rank = plsc.cumsum(msk.astype(jnp.int32))
                    plsc.store_scatter(
                        buf, [(cv + rank) - 1], iota + ch * _L, mask=msk
                    )
                    return cv + plsc.all_reduce_population_count(msk)

                return j + jnp.int32(1), cntv

            _, cntv = lax.while_loop(
                cond, step, (jnp.int32(0), jnp.zeros((_L,), jnp.int32))
            )
            lastv = jnp.maximum(cntv - 1, 0)
            sel0 = plsc.load_gather(buf, [jnp.minimum(iota, lastv)])
            sel1 = plsc.load_gather(buf, [jnp.minimum(iota + _L, lastv)])
            kt = lax.shift_right_logical(iota, 3)
            si = iota & 7
            pos = kt * (8 * qpw) + si * qpw + qi
            plsc.store_scatter(acc, [pos], sel0)
            plsc.store_scatter(acc, [pos + 2 * (8 * qpw)], sel1)
            return 0

        lax.fori_loop(0, qpw, per_query, 0)
        blk = 8 * qpw
        idx_cps = [
            pltpu.async_copy(
                acc.at[pl.ds(kt_i * blk, blk)],
                idx_hbm.at[pl.ds(b * total + kt_i * (M * 8) + slot * blk, blk)],
                sem_out,
            )
            for kt_i in range(_K // 8)
        ]
        for cp in idx_cps:
            cp.wait()

        plsc.subcore_barrier()

        prow = (px, py, pz, pn)
        for r in range(nrow):
            pltpu.sync_copy(idx_hbm.at[pl.ds(b * total + r * N, N)], prow[r])

        rows = (row0, row1)
        obs = (ob0, ob1)
        nxyz = 3

        pltpu.make_async_copy(feat_hbm.at[pl.ds(0, N)], row0, sem_pre).wait()

        nout = 0

        for cc in range(cpw):
            row = rows[cc % 2]
            ch = slot * cpw + cc
            obase = (b * CH + nxyz + ch) * total
            if cc == 1:
                pltpu.make_async_copy(
                    feat_hbm.at[pl.ds(0, N)], row, sem_pre
                ).wait()
            elif cc > 1:
                pltpu.make_async_copy(
                    feat_hbm.at[pl.ds(0, N)], row, sem_row
                ).wait()
            for ck in range(nch):
                ob = obs[ck % 2]
                irow = prow[ck // 2]
                ioff = (ck % 2) * chunk
                if nout >= 2:
                    pltpu.make_async_copy(ob, out.at[pl.ds(0, chunk)], sem_out).wait()
                    nout -= 1

                @plsc.parallel_loop(0, nvec, 1, unroll=8)
                def _g(j, _ioff=ioff, _ob=ob, _row=row, _irow=irow):
                    idxv = _irow[pl.ds(_ioff + j * _L, _L)]
                    _ob[pl.ds(j * _L, _L)] = plsc.load_gather(_row, [idxv])

                pltpu.async_copy(ob, out.at[pl.ds(obase + ck * chunk, chunk)], sem_out)
                nout += 1
            if cc + 2 < cpw:
                pltpu.async_copy(
                    feat_hbm.at[pl.ds((b * C + slot * cpw + cc + 2) * N, N)],
                    rows[cc % 2],
                    sem_row,
                )

        for _ in range(nout):
            pltpu.make_async_copy(ob0, out.at[pl.ds(0, chunk)], sem_out).wait()

        @pl.when(slot < nxyz)
        def _():
            pltpu.sync_copy(points_hbm.at[pl.ds((b * 3 + slot) * N, N)], row0)
            pltpu.sync_copy(newxyz_hbm.at[pl.ds((b * 3 + slot) * M, M)], ctr)
            obase = (b * CH + slot) * total
            for ck in range(nch):
                ob = obs[ck % 2]
                irow = prow[ck // 2]
                ioff = (ck % 2) * chunk
                if ck >= 2:
                    pltpu.make_async_copy(ob, out.at[pl.ds(0, chunk)], sem_out).wait()

                @plsc.parallel_loop(0, nvec, 1, unroll=8)
                def _g(j, _ck=ck, _ioff=ioff, _ob=ob, _irow=irow):
                    p = _ck * chunk + j * _L
                    idxv = _irow[pl.ds(_ioff + j * _L, _L)]
                    v = plsc.load_gather(row0, [idxv])
                    pv = iota + p
                    mv = ((lax.shift_right_logical(pv, 10) & 15) * 128) + (pv & 127)
                    cv = plsc.load_gather(ctr, [mv])
                    _ob[pl.ds(j * _L, _L)] = v - cv

                pltpu.async_copy(ob, out.at[pl.ds(obase + ck * chunk, chunk)], sem_out)
            for _ in range(min(nch, 2)):
                pltpu.make_async_copy(ob0, out.at[pl.ds(0, chunk)], sem_out).wait()

    points_i = lax.bitcast_convert_type(points_f, jnp.int32)
    return kern(points_f, points_i, newxyz_f, features_f)


def kernel(points_xyz, new_xyz, features):
    B, N, _ = points_xyz.shape
    M = new_xyz.shape[1]
    C = features.shape[1]
    points_f = jnp.transpose(points_xyz, (0, 2, 1)).reshape(-1)
    newxyz_f = jnp.transpose(new_xyz, (0, 2, 1)).reshape(-1)
    features_f = features.reshape(-1)
    _, out = _fused(points_f, newxyz_f, features_f, B, C, N, M)
    out5 = out.reshape(B, C + 3, _K // 8, M // 128, 8, 128)
    return jnp.transpose(out5, (0, 1, 3, 5, 2, 4)).reshape(B, C + 3, M, _K)
